# baseline (speedup 1.0000x reference)
import functools

import jax
import jax.numpy as jnp
from jax import lax
from jax.experimental import pallas as pl
from jax.experimental.pallas import tpu as pltpu
from jax.experimental.pallas import tpu_sc as plsc

_B = 1024
_D = 1024
_NP = 16
_NK = 32
_NL = 4
_ROW = _NL * _D

_NW = 32
_RPW = _B * _NP // _NW
_C = 8
_NCH = _RPW // _C


_BQ = 128


def _topk_body(q_ref, k_ref, idx_ref):
    q = q_ref[...]
    k = k_ref[...]
    dots = lax.dot_general(q, k, (((1,), (1,)), ((), ())),
                           preferred_element_type=jnp.float32)
    kn = jnp.sum(k * k, axis=1)
    score = dots * lax.rsqrt(kn)[None, :]
    score = jnp.where(score != score, jnp.inf, score)
    s_j = score[:, :, None]
    s_m = score[:, None, :]
    j_id = lax.broadcasted_iota(jnp.int32, (_BQ, _NK, _NK), 1)
    m_id = lax.broadcasted_iota(jnp.int32, (_BQ, _NK, _NK), 2)
    beats = (s_m > s_j) | ((s_m == s_j) & (m_id < j_id))
    rank = jnp.sum(beats.astype(jnp.int32), axis=2)
    r_id = lax.broadcasted_iota(jnp.int32, (_BQ, _NP, _NK), 1)
    j_id2 = lax.broadcasted_iota(jnp.int32, (_BQ, _NP, _NK), 2)
    onehot = rank[:, None, :] == r_id
    idx_ref[...] = jnp.sum(jnp.where(onehot, j_id2, 0), axis=2)


_topk_call = pl.pallas_call(
    _topk_body,
    grid=(_B // _BQ,),
    in_specs=[
        pl.BlockSpec((_BQ, _D), lambda i: (i, 0)),
        pl.BlockSpec((_NK, _D), lambda i: (0, 0)),
    ],
    out_specs=pl.BlockSpec((_BQ, _NP), lambda i: (i, 0)),
    out_shape=jax.ShapeDtypeStruct((_B, _NP), jnp.int32),
)


def _gather_body(pool_hbm, idx_hbm, out_hbm, idx_v, buf0, buf1, g0, g1):
    nc = 2
    wid = lax.axis_index("s") * nc + lax.axis_index("c")
    base = wid * _RPW
    pltpu.sync_copy(idx_hbm.at[pl.ds(base, _RPW)], idx_v)

    def fire(chunk, buf, sem):
        src = pool_hbm.at[idx_v.at[pl.ds(chunk * _C, _C)]]
        return pltpu.async_copy(src, buf, sem)

    def drain(buf, sem):
        pltpu.make_async_copy(pool_hbm.at[pl.ds(0, _C)], buf, sem).wait()

    def write(chunk, buf):
        pltpu.sync_copy(buf, out_hbm.at[pl.ds(base + chunk * _C, _C)])

    fire(0, buf0, g0)

    def body(t, carry):
        ch0 = 2 * t
        ch1 = ch0 + 1
        h1 = fire(ch1, buf1, g1)
        drain(buf0, g0)
        write(ch0, buf0)

        @pl.when(t < _NCH // 2 - 1)
        def _():
            fire(ch0 + 2, buf0, g0)

        h1.wait()
        write(ch1, buf1)
        return carry

    lax.fori_loop(0, _NCH // 2, body, 0)


@functools.cache
def _make_gather_call():
    mesh = plsc.VectorSubcoreMesh(core_axis_name="c", subcore_axis_name="s")
    return pl.kernel(
        _gather_body,
        out_type=jax.ShapeDtypeStruct((_B * _NP, _ROW), jnp.float32),
        mesh=mesh,
        scratch_types=[
            pltpu.VMEM((_RPW,), jnp.int32),
            pltpu.VMEM((_C, _ROW), jnp.float32),
            pltpu.VMEM((_C, _ROW), jnp.float32),
            pltpu.SemaphoreType.DMA,
            pltpu.SemaphoreType.DMA,
        ],
    )


def kernel(query, prompt_pool, prompt_key):
    idx = _topk_call(query, prompt_key)
    flat = _make_gather_call()(prompt_pool, idx.reshape(-1))
    return flat.reshape(_B, _NP * _NL, _D)


# speedup vs baseline: 1.9915x; 1.9915x over previous
import functools

import jax
import jax.numpy as jnp
from jax import lax
from jax.experimental import pallas as pl
from jax.experimental.pallas import tpu as pltpu
from jax.experimental.pallas import tpu_sc as plsc

_B = 1024
_D = 1024
_NP = 16
_NK = 32
_NL = 4
_Q = _NP * _NL
_PR = _NK * _NL

_NW = 32
_BPW = _B // _NW
_C = 16
_NBUF = 4
_NCH = _BPW * _Q // _C
_HPB = _Q // _C


def _topk_body(q_ref, k_ref, idx_ref):
    q = q_ref[...]
    k = k_ref[...]
    dots = lax.dot_general(k, q, (((1,), (1,)), ((), ())),
                           preferred_element_type=jnp.float32)
    kn = jnp.sum(k * k, axis=1, keepdims=True)
    score = dots * lax.rsqrt(kn)
    score = jnp.where(score != score, jnp.inf, score)
    m_id = lax.broadcasted_iota(jnp.int32, (_NK, _B), 0)
    rank_rows = []
    for j in range(_NK):
        sj = score[j:j + 1, :]
        beats = (score > sj) | ((score == sj) & (m_id < j))
        rank_rows.append(jnp.sum(beats.astype(jnp.int32), axis=0,
                                 keepdims=True))
    rank = jnp.concatenate(rank_rows, axis=0)
    row4 = 4 * m_id
    for qq in range(_Q):
        sel = jnp.sum(jnp.where(rank == qq // _NL, row4 + (qq % _NL), 0),
                      axis=0, keepdims=True)
        idx_ref[qq:qq + 1, :] = sel


_topk_call = pl.pallas_call(
    _topk_body,
    out_shape=jax.ShapeDtypeStruct((_Q, _B), jnp.int32),
)


def _gather_body(pool_hbm, idx_hbm, out_hbm, idx_v, buf, gsems, wsems):
    nc = 2
    wid = lax.axis_index("s") * nc + lax.axis_index("c")
    base = wid * _BPW * _Q
    pltpu.sync_copy(idx_hbm.at[pl.ds(base, _BPW * _Q)], idx_v)

    def fire_gather(chunk, s):
        src = pool_hbm.at[idx_v.at[pl.ds(chunk * _C, _C)]]
        return pltpu.async_copy(src, buf.at[s], gsems[s])

    def wait_gather(s):
        pltpu.make_async_copy(pool_hbm.at[pl.ds(0, _C)], buf.at[s],
                              gsems[s]).wait()

    def fire_write(chunk, s):
        b = wid * _BPW + chunk // _HPB
        h = chunk % _HPB
        dst = out_hbm.at[b, pl.ds(h * _C, _C)]
        return pltpu.async_copy(buf.at[s], dst, wsems[s])

    def wait_write(s):
        pltpu.make_async_copy(pool_hbm.at[pl.ds(0, _C)], buf.at[s],
                              wsems[s]).wait()

    for s in range(_NBUF):
        fire_gather(s, s)

    def body(g, carry):
        for s in range(_NBUF):
            t = g * _NBUF + s
            wait_gather(s)
            fire_write(t, s)

            @pl.when(g < _NCH // _NBUF - 1)
            def _():
                wait_write(s)
                fire_gather(t + _NBUF, s)

        return carry

    lax.fori_loop(0, _NCH // _NBUF, body, 0)
    for s in range(_NBUF):
        wait_write(s)


@functools.cache
def _make_gather_call():
    mesh = plsc.VectorSubcoreMesh(core_axis_name="c", subcore_axis_name="s")
    return pl.kernel(
        _gather_body,
        out_type=jax.ShapeDtypeStruct((_B, _Q, _D), jnp.float32),
        mesh=mesh,
        scratch_types=[
            pltpu.VMEM((_BPW * _Q,), jnp.int32),
            pltpu.VMEM((_NBUF, _C, _D), jnp.float32),
            [pltpu.SemaphoreType.DMA] * _NBUF,
            [pltpu.SemaphoreType.DMA] * _NBUF,
        ],
    )


def kernel(query, prompt_pool, prompt_key):
    idx4_t = _topk_call(query, prompt_key)
    idx4 = idx4_t.T.reshape(-1)
    pool_rows = prompt_pool.reshape(_PR, _D)
    return _make_gather_call()(pool_rows, idx4)


# speedup vs baseline: 3.3072x; 1.6607x over previous
import functools

import jax
import jax.numpy as jnp
from jax import lax
from jax.experimental import pallas as pl
from jax.experimental.pallas import tpu as pltpu
from jax.experimental.pallas import tpu_sc as plsc

_B = 1024
_D = 1024
_NP = 16
_NK = 32
_NL = 4
_Q = _NP * _NL
_PR = _NK * _NL

_NW = 32
_BPW = _B // _NW
_C = 16
_NBUF = 4
_NCH = _BPW * _Q // _C
_HPB = _Q // _C


def _topk_body(q_ref, k_ref, idx_ref):
    q = q_ref[...]
    k = k_ref[...]
    dots = lax.dot_general(k, q, (((1,), (1,)), ((), ())),
                           preferred_element_type=jnp.float32)
    kn = jnp.sum(k * k, axis=1, keepdims=True)
    score = dots * lax.rsqrt(kn)
    score = jnp.where(score != score, jnp.inf, score)
    m_id = lax.broadcasted_iota(jnp.int32, (_NK, _B), 0)
    rank_rows = []
    for j in range(_NK):
        sj = score[j:j + 1, :]
        beats = (score > sj) | ((score == sj) & (m_id < j))
        rank_rows.append(jnp.sum(beats.astype(jnp.int32), axis=0,
                                 keepdims=True))
    rank = jnp.concatenate(rank_rows, axis=0)
    b_id = lax.broadcasted_iota(jnp.int32, (1, _B), 1)
    repl_off = (b_id // _BPW) * _PR
    row4 = 4 * m_id
    for qq in range(_Q):
        sel = jnp.sum(jnp.where(rank == qq // _NL, row4 + (qq % _NL), 0),
                      axis=0, keepdims=True)
        idx_ref[qq:qq + 1, :] = sel + repl_off


_topk_call = pl.pallas_call(
    _topk_body,
    out_shape=jax.ShapeDtypeStruct((_Q, _B), jnp.int32),
)


def _gather_body(pool_hbm, idx_hbm, out_hbm, idx_v, buf, gsems, wsems):
    nc = 2
    wid = lax.axis_index("s") * nc + lax.axis_index("c")
    base = wid * _BPW * _Q
    pltpu.sync_copy(idx_hbm.at[pl.ds(base, _BPW * _Q)], idx_v)

    def fire_gather(chunk, s):
        src = pool_hbm.at[idx_v.at[pl.ds(chunk * _C, _C)]]
        return pltpu.async_copy(src, buf.at[s], gsems[s])

    def wait_gather(s):
        pltpu.make_async_copy(pool_hbm.at[pl.ds(0, _C)], buf.at[s],
                              gsems[s]).wait()

    def fire_write(chunk, s):
        b = wid * _BPW + chunk // _HPB
        h = chunk % _HPB
        dst = out_hbm.at[b, pl.ds(h * _C, _C)]
        return pltpu.async_copy(buf.at[s], dst, wsems[s])

    def wait_write(s):
        pltpu.make_async_copy(pool_hbm.at[pl.ds(0, _C)], buf.at[s],
                              wsems[s]).wait()

    for s in range(_NBUF):
        fire_gather(s, s)

    def body(g, carry):
        for s in range(_NBUF):
            t = g * _NBUF + s
            wait_gather(s)
            fire_write(t, s)

            @pl.when(g < _NCH // _NBUF - 1)
            def _():
                wait_write(s)
                fire_gather(t + _NBUF, s)

        return carry

    lax.fori_loop(0, _NCH // _NBUF, body, 0)
    for s in range(_NBUF):
        wait_write(s)


@functools.cache
def _make_gather_call():
    mesh = plsc.VectorSubcoreMesh(core_axis_name="c", subcore_axis_name="s")
    return pl.kernel(
        _gather_body,
        out_type=jax.ShapeDtypeStruct((_B, _Q, _D), jnp.float32),
        mesh=mesh,
        scratch_types=[
            pltpu.VMEM((_BPW * _Q,), jnp.int32),
            pltpu.VMEM((_NBUF, _C, _D), jnp.float32),
            [pltpu.SemaphoreType.DMA] * _NBUF,
            [pltpu.SemaphoreType.DMA] * _NBUF,
        ],
    )


def kernel(query, prompt_pool, prompt_key):
    idx4_t = _topk_call(query, prompt_key)
    idx4 = idx4_t.T.reshape(-1)
    pool_rows = prompt_pool.reshape(_PR, _D)
    pool_rep = jnp.tile(pool_rows, (_NW, 1))
    return _make_gather_call()(pool_rep, idx4)


# speedup vs baseline: 5.8185x; 1.7593x over previous
import functools

import jax
import jax.numpy as jnp
from jax import lax
from jax.experimental import pallas as pl
from jax.experimental.pallas import tpu as pltpu
from jax.experimental.pallas import tpu_sc as plsc

_B = 1024
_D = 1024
_NP = 16
_NK = 32
_NL = 4
_Q = _NP * _NL
_PL = (_NK - 1) * _NL + 1

_NW = 32
_BPW = _B // _NW
_C = 16
_NSEM = 8
_NCH = _BPW * _Q // _C
_HPB = _Q // _C


def _topk_body(q_ref, k_ref, idx_ref):
    q = q_ref[...]
    k = k_ref[...]
    dots = lax.dot_general(k, q, (((1,), (1,)), ((), ())),
                           preferred_element_type=jnp.float32)
    kn = jnp.sum(k * k, axis=1, keepdims=True)
    score = dots * lax.rsqrt(kn)
    score = jnp.where(score != score, jnp.inf, score)
    m_id = lax.broadcasted_iota(jnp.int32, (_NK, _B), 0)
    rank_rows = []
    for j in range(_NK):
        sj = score[j:j + 1, :]
        beats = (score > sj) | ((score == sj) & (m_id < j))
        rank_rows.append(jnp.sum(beats.astype(jnp.int32), axis=0,
                                 keepdims=True))
    rank = jnp.concatenate(rank_rows, axis=0)
    row4 = 4 * m_id
    for qq in range(_Q):
        local = jnp.where(m_id == 0, 0, row4 + (qq % _NL - 3))
        sel = jnp.sum(jnp.where(rank == qq // _NL, local, 0),
                      axis=0, keepdims=True)
        idx_ref[qq:qq + 1, :] = sel


_topk_call = pl.pallas_call(
    _topk_body,
    out_shape=jax.ShapeDtypeStruct((_Q, _B), jnp.int32),
)


def _gather_body(pool_hbm, idx_hbm, out_hbm, idx_v, pool_v, wsems):
    nc = 2
    wid = lax.axis_index("s") * nc + lax.axis_index("c")
    pltpu.sync_copy(idx_hbm.at[pl.ds(wid * _BPW * _Q, _BPW * _Q)], idx_v)
    pltpu.sync_copy(pool_hbm.at[pl.ds(wid * _PL * _D, _PL * _D)], pool_v)

    def wait_write(s):
        pltpu.make_async_copy(pool_hbm.at[pl.ds(0, _D)],
                              pool_v.at[pl.ds(0, _D)], wsems[s]).wait()

    def body(g, carry):
        rows = idx_v[pl.ds(g * 16, 16)]
        for l in range(16):
            s = l % _NSEM
            if l < _NSEM:
                @pl.when(g > 0)
                def _():
                    wait_write(s)
            else:
                wait_write(s)

            r = g * 16 + l
            row = rows[l]
            b = wid * _BPW + r // _Q
            q = r % _Q
            src = pool_v.at[pl.ds(row * _D, _D)]
            dst = out_hbm.at[b, q]
            pltpu.async_copy(src, dst, wsems[s])
        return carry

    lax.fori_loop(0, _BPW * _Q // 16, body, 0)
    for s in range(_NSEM):
        wait_write(s)


@functools.cache
def _make_gather_call():
    mesh = plsc.VectorSubcoreMesh(core_axis_name="c", subcore_axis_name="s")
    return pl.kernel(
        _gather_body,
        out_type=jax.ShapeDtypeStruct((_B, _Q, _D), jnp.float32),
        mesh=mesh,
        scratch_types=[
            pltpu.VMEM((_BPW * _Q,), jnp.int32),
            pltpu.VMEM((_PL * _D,), jnp.float32),
            [pltpu.SemaphoreType.DMA] * _NSEM,
        ],
    )


def kernel(query, prompt_pool, prompt_key):
    idx4_t = _topk_call(query, prompt_key)
    idx4 = idx4_t.T.reshape(-1)
    pool_rows = prompt_pool.reshape(_NK * _NL, _D)
    pool_c = jnp.concatenate(
        [jnp.zeros((1, _D), jnp.float32), pool_rows[_NL:]], axis=0)
    pool_rep = jnp.tile(pool_c, (_NW, 1)).reshape(-1)
    return _make_gather_call()(pool_rep, idx4)


# speedup vs baseline: 5.8970x; 1.0135x over previous
import functools

import jax
import jax.numpy as jnp
from jax import lax
from jax.experimental import pallas as pl
from jax.experimental.pallas import tpu as pltpu
from jax.experimental.pallas import tpu_sc as plsc

_B = 1024
_D = 1024
_NP = 16
_NK = 32
_NL = 4
_Q = _NP * _NL
_PL = (_NK - 1) * _NL + 1

_NW = 32
_BPW = _B // _NW
_C = 16
_NSEM = 8
_NCH = _BPW * _Q // _C
_HPB = _Q // _C


def _topk_body(q_ref, k_ref, idx_ref):
    q = q_ref[...]
    k = k_ref[...]
    dots = lax.dot_general(k, q, (((1,), (1,)), ((), ())),
                           preferred_element_type=jnp.float32)
    kn = jnp.sum(k * k, axis=1, keepdims=True)
    score = dots * lax.rsqrt(kn)
    score = jnp.where(score != score, jnp.inf, score)
    m_id = lax.broadcasted_iota(jnp.int32, (_NK, _B), 0)
    rank_rows = []
    for j in range(_NK):
        sj = score[j:j + 1, :]
        beats = (score > sj) | ((score == sj) & (m_id < j))
        rank_rows.append(jnp.sum(beats.astype(jnp.int32), axis=0,
                                 keepdims=True))
    rank = jnp.concatenate(rank_rows, axis=0)
    row4 = 4 * m_id
    for qq in range(_Q):
        local = jnp.where(m_id == 0, 0, row4 + (qq % _NL - 3))
        sel = jnp.sum(jnp.where(rank == qq // _NL, local, 0),
                      axis=0, keepdims=True)
        idx_ref[qq:qq + 1, :] = sel


_topk_call = pl.pallas_call(
    _topk_body,
    out_shape=jax.ShapeDtypeStruct((_Q, _B), jnp.int32),
)


def _gather_body(pool_hbm, idx_hbm, out_hbm, idx_v, pool_v, wsems):
    nc = 2
    wid = lax.axis_index("s") * nc + lax.axis_index("c")
    pltpu.sync_copy(idx_hbm.at[pl.ds(wid * _BPW * _Q, _BPW * _Q)], idx_v)
    pltpu.sync_copy(pool_hbm.at[pl.ds(0, _PL * _D)], pool_v)

    def wait_write(s):
        pltpu.make_async_copy(pool_hbm.at[pl.ds(0, _D)],
                              pool_v.at[pl.ds(0, _D)], wsems[s]).wait()

    def body(g, carry):
        rows = idx_v[pl.ds(g * 16, 16)]
        for l in range(16):
            s = l % _NSEM
            if l < _NSEM:
                @pl.when(g > 0)
                def _():
                    wait_write(s)
            else:
                wait_write(s)

            r = g * 16 + l
            row = rows[l]
            b = wid * _BPW + r // _Q
            q = r % _Q
            src = pool_v.at[pl.ds(row * _D, _D)]
            dst = out_hbm.at[b, q]
            pltpu.async_copy(src, dst, wsems[s])
        return carry

    lax.fori_loop(0, _BPW * _Q // 16, body, 0)
    for s in range(_NSEM):
        wait_write(s)


@functools.cache
def _make_gather_call():
    mesh = plsc.VectorSubcoreMesh(core_axis_name="c", subcore_axis_name="s")
    return pl.kernel(
        _gather_body,
        out_type=jax.ShapeDtypeStruct((_B, _Q, _D), jnp.float32),
        mesh=mesh,
        scratch_types=[
            pltpu.VMEM((_BPW * _Q,), jnp.int32),
            pltpu.VMEM((_PL * _D,), jnp.float32),
            [pltpu.SemaphoreType.DMA] * _NSEM,
        ],
    )


def kernel(query, prompt_pool, prompt_key):
    idx4_t = _topk_call(query, prompt_key)
    idx4 = idx4_t.T.reshape(-1)
    pool_rows = prompt_pool.reshape(_NK * _NL, _D)
    pool_c = jnp.concatenate(
        [jnp.zeros((1, _D), jnp.float32), pool_rows[_NL:]], axis=0)
    return _make_gather_call()(pool_c.reshape(-1), idx4)
